# baseline (speedup 1.0000x reference)
import functools

import jax
import jax.numpy as jnp
from jax import lax
from jax.experimental import pallas as pl
from jax.experimental.pallas import tpu as pltpu
from jax.experimental.pallas import tpu_sc as plsc

NC, NS, L = 2, 16, 16
NW = NC * NS

CHUNK = 64


def _leaky(x):
    return jnp.maximum(x, 0.01 * x)


def kernel(state, action, emb_table):
    B, SD = state.shape
    _, ED = emb_table.shape
    OD = SD + ED
    rows_per_w = B // NW
    nchunk = rows_per_w // CHUNK

    mesh = plsc.VectorSubcoreMesh(
        core_axis_name="c", subcore_axis_name="s", num_cores=NC, num_subcores=NS
    )

    @functools.partial(
        pl.kernel,
        out_type=jax.ShapeDtypeStruct((B * OD,), jnp.float32),
        mesh=mesh,
        scratch_types=[
            pltpu.VMEM((CHUNK,), jnp.int32),
            pltpu.VMEM((CHUNK * SD,), jnp.float32),
            pltpu.VMEM((CHUNK, ED), jnp.float32),
            pltpu.VMEM((CHUNK * OD,), jnp.float32),
            pltpu.SemaphoreType.DMA,
        ],
        compiler_params=pltpu.CompilerParams(use_tc_tiling_on_sc=False),
    )
    def sc_kernel(state_hbm, action_hbm, table_hbm, out_hbm, idx_v, st_v, emb_v, out_v, sem):
        wid = lax.axis_index("s") * NC + lax.axis_index("c")
        w_base = wid * rows_per_w

        def chunk_body(ci, carry):
            base = w_base + ci * CHUNK
            pltpu.sync_copy(action_hbm.at[pl.ds(base, CHUNK)], idx_v)
            pltpu.async_copy(table_hbm.at[idx_v], emb_v, sem).wait()
            pltpu.sync_copy(state_hbm.at[pl.ds(base * SD, CHUNK * SD)], st_v)

            def row_body(r, c2):
                for j in range(SD // L):
                    x = st_v[pl.ds(r * SD + j * L, L)]
                    out_v[pl.ds(r * OD + j * L, L)] = _leaky(x)
                for j in range(ED // L):
                    x = emb_v[r, pl.ds(j * L, L)]
                    out_v[pl.ds(r * OD + SD + j * L, L)] = _leaky(x)
                return c2

            lax.fori_loop(0, CHUNK, row_body, 0)
            pltpu.sync_copy(out_v, out_hbm.at[pl.ds(base * OD, CHUNK * OD)])
            return carry

        lax.fori_loop(0, nchunk, chunk_body, 0)

    out = sc_kernel(state.reshape(-1), action.astype(jnp.int32), emb_table)
    return out.reshape(B, OD)


# speedup vs baseline: 1.5988x; 1.5988x over previous
import functools

import jax
import jax.numpy as jnp
from jax import lax
from jax.experimental import pallas as pl
from jax.experimental.pallas import tpu as pltpu
from jax.experimental.pallas import tpu_sc as plsc

NC, NS, L = 2, 16, 16
NW = NC * NS

CHUNK = 64


def _leaky(x):
    return jnp.maximum(x, 0.01 * x)


def kernel(state, action, emb_table):
    B, SD = state.shape
    _, ED = emb_table.shape
    OD = SD + ED
    rows_per_w = B // NW
    nchunk = rows_per_w // CHUNK

    mesh = plsc.VectorSubcoreMesh(
        core_axis_name="c", subcore_axis_name="s", num_cores=NC, num_subcores=NS
    )

    @functools.partial(
        pl.kernel,
        out_type=jax.ShapeDtypeStruct((B, OD), jnp.float32),
        mesh=mesh,
        scratch_types=[
            pltpu.VMEM((rows_per_w,), jnp.int32),
            pltpu.VMEM((CHUNK, SD), jnp.float32),
            pltpu.VMEM((CHUNK, ED), jnp.float32),
            pltpu.VMEM((CHUNK, OD), jnp.float32),
            pltpu.SemaphoreType.DMA,
        ],
        compiler_params=pltpu.CompilerParams(use_tc_tiling_on_sc=True),
    )
    def sc_kernel(state_hbm, action_hbm, table_hbm, out_hbm, idx_v, st_v, emb_v, out_v, sem):
        wid = lax.axis_index("s") * NC + lax.axis_index("c")
        w_base = wid * rows_per_w
        pltpu.sync_copy(action_hbm.at[pl.ds(w_base, rows_per_w)], idx_v)

        def chunk_body(ci, carry):
            base = w_base + ci * CHUNK
            pltpu.sync_copy(state_hbm.at[pl.ds(base, CHUNK), :], st_v)

            def fire(g, c2):
                vec = idx_v[pl.ds(ci * CHUNK + g * L, L)]
                for lane in range(L):
                    row = vec[lane]
                    pltpu.async_copy(table_hbm.at[row], emb_v.at[g * L + lane], sem)
                return c2

            lax.fori_loop(0, CHUNK // L, fire, 0)
            pltpu.make_async_copy(table_hbm.at[pl.ds(0, CHUNK), :], emb_v, sem).wait()

            def row_body(r, c2):
                for j in range(SD // L):
                    x = st_v[r, pl.ds(j * L, L)]
                    out_v[r, pl.ds(j * L, L)] = _leaky(x)
                for j in range(ED // L):
                    x = emb_v[r, pl.ds(j * L, L)]
                    out_v[r, pl.ds(SD + j * L, L)] = _leaky(x)
                return c2

            lax.fori_loop(0, CHUNK, row_body, 0)
            pltpu.sync_copy(out_v, out_hbm.at[pl.ds(base, CHUNK), :])
            return carry

        lax.fori_loop(0, nchunk, chunk_body, 0)

    return sc_kernel(state, action.astype(jnp.int32), emb_table)


# speedup vs baseline: 2.1598x; 1.3509x over previous
import functools

import jax
import jax.numpy as jnp
from jax import lax
from jax.experimental import pallas as pl
from jax.experimental.pallas import tpu as pltpu
from jax.experimental.pallas import tpu_sc as plsc

NC, NS, L = 2, 16, 16
NW = NC * NS

BN = 2048


def _leaky(x):
    return jnp.maximum(x, 0.01 * x)


def _sc_gather_rows(table, act):
    _, ED = table.shape
    (B,) = act.shape
    rpw = B // NW

    mesh = plsc.VectorSubcoreMesh(
        core_axis_name="c", subcore_axis_name="s", num_cores=NC, num_subcores=NS
    )

    @functools.partial(
        pl.kernel,
        out_type=jax.ShapeDtypeStruct((B, ED), jnp.float32),
        mesh=mesh,
        scratch_types=[
            pltpu.VMEM((rpw,), jnp.int32),
            pltpu.VMEM((rpw, ED), jnp.float32),
            pltpu.SemaphoreType.DMA,
        ],
        compiler_params=pltpu.CompilerParams(use_tc_tiling_on_sc=True),
    )
    def gather_kernel(table_hbm, act_hbm, out_hbm, idx_v, emb_v, sem):
        wid = lax.axis_index("s") * NC + lax.axis_index("c")
        b0 = wid * rpw
        pltpu.sync_copy(act_hbm.at[pl.ds(b0, rpw)], idx_v)

        def fire(g, carry):
            vec = idx_v[pl.ds(g * L, L)]
            for lane in range(L):
                r = vec[lane]
                pltpu.async_copy(table_hbm.at[r], emb_v.at[g * L + lane], sem)
            return carry

        lax.fori_loop(0, rpw // L, fire, 0)
        pltpu.make_async_copy(table_hbm.at[pl.ds(0, rpw)], emb_v, sem).wait()
        pltpu.sync_copy(emb_v, out_hbm.at[pl.ds(b0, rpw)])

    return gather_kernel(table, act)


def _tc_assemble(state, emb_raw):
    B, SD = state.shape
    _, ED = emb_raw.shape
    OD = SD + ED

    def body(st_ref, emb_ref, out_ref):
        out_ref[pl.ds(0, SD), :] = _leaky(st_ref[...].T)
        out_ref[pl.ds(SD, ED), :] = _leaky(emb_ref[...].T)

    return pl.pallas_call(
        body,
        grid=(B // BN,),
        in_specs=[
            pl.BlockSpec((BN, SD), lambda i: (i, 0)),
            pl.BlockSpec((BN, ED), lambda i: (i, 0)),
        ],
        out_specs=pl.BlockSpec((OD, BN), lambda i: (0, i)),
        out_shape=jax.ShapeDtypeStruct((OD, B), jnp.float32),
    )(state, emb_raw)


def kernel(state, action, emb_table):
    act = action.astype(jnp.int32)
    emb_raw = _sc_gather_rows(emb_table, act)
    out_t = _tc_assemble(state, emb_raw)
    return out_t.T
